# Initial kernel scaffold; baseline (speedup 1.0000x reference)
import jax
import jax.numpy as jnp
from jax.experimental import pallas as pl


def kernel(child_info, emb_sex, emb_b_month):
    raise NotImplementedError("write your pallas kernel here")


# speedup vs baseline: 1.1071x; 1.1071x over previous
import jax
import jax.numpy as jnp
from jax import lax
from jax.experimental import pallas as pl
from jax.experimental.pallas import tpu as pltpu
from jax.experimental.pallas import tpu_sc as plsc

_B = 16384
_DO = 256
_NC = 2
_NS = 16
_NW = _NC * _NS
_BW = _B // _NW
_CH = 128
_NCHUNK = _BW // _CH
_L = 16


def _tile_body(sex_hbm, bm_hbm, comb_hbm, out_hbm, sex_v, bm_v, idx_v,
               rows_v, gsems):
    wid = lax.axis_index("s") * _NC + lax.axis_index("c")
    base = wid * _BW
    pltpu.sync_copy(sex_hbm.at[pl.ds(base, _BW)], sex_v)
    pltpu.sync_copy(bm_hbm.at[pl.ds(base, _BW)], bm_v)
    for i in range(_BW // _L):
        sl = pl.ds(i * _L, _L)
        idx_v[sl] = sex_v[sl] * 13 + bm_v[sl]

    def gather(j):
        return pltpu.async_copy(
            comb_hbm.at[idx_v.at[pl.ds(j * _CH, _CH)]],
            rows_v.at[j % 2],
            gsems.at[j % 2],
        )

    cps = [None, None]
    cps[0] = gather(0)
    for j in range(_NCHUNK):
        if j + 1 < _NCHUNK:
            cps[(j + 1) % 2] = gather(j + 1)
        cps[j % 2].wait()
        pltpu.sync_copy(rows_v.at[j % 2],
                        out_hbm.at[pl.ds(base + j * _CH, _CH)])


_lookup = pl.kernel(
    _tile_body,
    out_type=jax.ShapeDtypeStruct((_B, _DO), jnp.float32),
    mesh=plsc.VectorSubcoreMesh(core_axis_name="c", subcore_axis_name="s",
                                num_cores=_NC, num_subcores=_NS),
    scratch_types=[
        pltpu.VMEM((_BW,), jnp.int32),
        pltpu.VMEM((_BW,), jnp.int32),
        pltpu.VMEM((_BW,), jnp.int32),
        pltpu.VMEM((2, _CH, _DO), jnp.float32),
        pltpu.SemaphoreType.DMA((2,)),
    ],
)


def kernel(child_info, emb_sex, emb_b_month):
    info = child_info.astype(jnp.int32)
    comb = jnp.concatenate(
        [jnp.repeat(emb_sex, 13, axis=0), jnp.tile(emb_b_month, (3, 1))],
        axis=1,
    )
    return _lookup(info[:, 0], info[:, 1], comb)


# speedup vs baseline: 1.6696x; 1.5080x over previous
import jax
import jax.numpy as jnp
from jax import lax
from jax.experimental import pallas as pl
from jax.experimental.pallas import tpu as pltpu
from jax.experimental.pallas import tpu_sc as plsc

_B = 16384
_DO = 256
_NT = 39
_NC = 2
_NS = 16
_NW = _NC * _NS
_BW = _B // _NW
_CH = 128
_NCHUNK = _BW // _CH
_L = 16


def _tile_body(sex_hbm, bm_hbm, comb_hbm, out_hbm, sex_v, bm_v, idx_v,
               tab_v, rows_a, rows_b, osems):
    wid = lax.axis_index("s") * _NC + lax.axis_index("c")
    base = wid * _BW
    pltpu.sync_copy(sex_hbm.at[pl.ds(base, _BW)], sex_v)
    pltpu.sync_copy(bm_hbm.at[pl.ds(base, _BW)], bm_v)
    pltpu.sync_copy(comb_hbm, tab_v)
    for i in range(_BW // _L):
        sl = pl.ds(i * _L, _L)
        idx_v[sl] = (sex_v[sl] * 13 + bm_v[sl]) * _DO
    cps = [None, None]
    for j in range(_NCHUNK):
        buf = j % 2
        if cps[buf] is not None:
            cps[buf].wait()
        dst = (rows_a, rows_b)[buf]

        def group_body(g, _, j=j, dst=dst):
            iv = idx_v[pl.ds(j * _CH + g * _L, _L)]
            for l in range(_L):
                src = iv[l]
                rbase = (g * _L + l) * _DO
                for k in range(0, _DO, _L):
                    dst[pl.ds(rbase + k, _L)] = tab_v[pl.ds(src + k, _L)]
            return 0

        lax.fori_loop(0, _CH // _L, group_body, 0)
        cps[buf] = pltpu.async_copy(
            dst,
            out_hbm.at[pl.ds((base + j * _CH) * _DO, _CH * _DO)],
            osems.at[buf],
        )
    for cp in cps:
        cp.wait()


_lookup = pl.kernel(
    _tile_body,
    out_type=jax.ShapeDtypeStruct((_B * _DO,), jnp.float32),
    mesh=plsc.VectorSubcoreMesh(core_axis_name="c", subcore_axis_name="s",
                                num_cores=_NC, num_subcores=_NS),
    compiler_params=pltpu.CompilerParams(needs_layout_passes=False),
    scratch_types=[
        pltpu.VMEM((_BW,), jnp.int32),
        pltpu.VMEM((_BW,), jnp.int32),
        pltpu.VMEM((_BW,), jnp.int32),
        pltpu.VMEM((_NT * _DO,), jnp.float32),
        pltpu.VMEM((_CH * _DO,), jnp.float32),
        pltpu.VMEM((_CH * _DO,), jnp.float32),
        pltpu.SemaphoreType.DMA((2,)),
    ],
)


def kernel(child_info, emb_sex, emb_b_month):
    info = child_info.astype(jnp.int32)
    comb = jnp.concatenate(
        [jnp.repeat(emb_sex, 13, axis=0), jnp.tile(emb_b_month, (3, 1))],
        axis=1,
    ).reshape(-1)
    out = _lookup(info[:, 0], info[:, 1], comb)
    return out.reshape(_B, _DO)


# speedup vs baseline: 3.3312x; 1.9952x over previous
import jax
import jax.numpy as jnp
from jax import lax
from jax.experimental import pallas as pl
from jax.experimental.pallas import tpu as pltpu
from jax.experimental.pallas import tpu_sc as plsc

_B = 16384
_DO = 256
_NT = 39
_NC = 2
_NS = 16
_NW = _NC * _NS
_BW = _B // _NW
_CH = 128
_NCHUNK = _BW // _CH
_L = 16


def _tile_body(sex_hbm, bm_hbm, comb_hbm, out_hbm, sex_v, bm_v, idx_v,
               tab_v, rows_a, rows_b, osems):
    wid = lax.axis_index("s") * _NC + lax.axis_index("c")
    base = wid * _BW
    pltpu.sync_copy(sex_hbm.at[pl.ds(base, _BW)], sex_v)
    pltpu.sync_copy(bm_hbm.at[pl.ds(base, _BW)], bm_v)
    pltpu.sync_copy(comb_hbm, tab_v)
    for i in range(_BW // _L):
        sl = pl.ds(i * _L, _L)
        idx_v[sl] = (sex_v[sl] * 13 + bm_v[sl]) * _DO
    cps = [None, None]
    for j in range(_NCHUNK):
        buf = j % 2
        if cps[buf] is not None:
            cps[buf].wait()
        dst = (rows_a, rows_b)[buf]

        def group_body(g, _, j=j, dst=dst):
            iv = idx_v[pl.ds(j * _CH + g * _L, _L)]
            srcs = [iv[l] for l in range(_L)]
            for l in range(_L):
                vs = [tab_v[pl.ds(srcs[l] + k, _L)]
                      for k in range(0, _DO, _L)]
                for k, v in enumerate(vs):
                    dst[g * _L + l, pl.ds(k * _L, _L)] = v
            return 0

        lax.fori_loop(0, _CH // _L, group_body, 0)
        cps[buf] = pltpu.async_copy(
            dst,
            out_hbm.at[pl.ds(base + j * _CH, _CH)],
            osems.at[buf],
        )
    for cp in cps:
        cp.wait()


_lookup = pl.kernel(
    _tile_body,
    out_type=jax.ShapeDtypeStruct((_B, _DO), jnp.float32),
    mesh=plsc.VectorSubcoreMesh(core_axis_name="c", subcore_axis_name="s",
                                num_cores=_NC, num_subcores=_NS),
    compiler_params=pltpu.CompilerParams(needs_layout_passes=False),
    scratch_types=[
        pltpu.VMEM((_BW,), jnp.int32),
        pltpu.VMEM((_BW,), jnp.int32),
        pltpu.VMEM((_BW,), jnp.int32),
        pltpu.VMEM((_NT * _DO,), jnp.float32),
        pltpu.VMEM((_CH, _DO), jnp.float32),
        pltpu.VMEM((_CH, _DO), jnp.float32),
        pltpu.SemaphoreType.DMA((2,)),
    ],
)


def kernel(child_info, emb_sex, emb_b_month):
    info = child_info.astype(jnp.int32)
    comb = jnp.concatenate(
        [jnp.repeat(emb_sex, 13, axis=0), jnp.tile(emb_b_month, (3, 1))],
        axis=1,
    ).reshape(-1)
    return _lookup(info[:, 0], info[:, 1], comb)
